# baseline (speedup 1.0000x reference)
import functools
import math

import jax
import jax.numpy as jnp
from jax import lax
from jax.experimental import pallas as pl
from jax.experimental.pallas import tpu as pltpu
from jax.experimental.pallas import tpu_sc as plsc

EMB_D = 64
SCALE = math.sqrt(EMB_D)

NUM_CORES = 2
NUM_SUBCORES = 16
NW = NUM_CORES * NUM_SUBCORES

IDX_BLK = 128
NJ = 4
CHUNK = NJ * IDX_BLK


def _emb_kernel_body(n_chunks, tok_hbm, table_hbm, out_hbm, idx_v, rows_v, sem):
    wid = lax.axis_index("s") * NUM_CORES + lax.axis_index("c")
    rows_per_w = n_chunks * CHUNK
    row0 = wid * (rows_per_w // IDX_BLK)

    def chunk_body(g, carry):
        pltpu.sync_copy(tok_hbm.at[pl.ds(row0 + g * NJ, NJ)], idx_v)
        descs = []
        for j in range(NJ):
            descs.append(
                pltpu.async_copy(
                    table_hbm.at[idx_v.at[j]],
                    rows_v.at[pl.ds(j * IDX_BLK, IDX_BLK)],
                    sem,
                )
            )
        for d in descs:
            d.wait()

        def scale_row(i, c):
            for j in range(EMB_D // 16):
                rows_v[i, pl.ds(j * 16, 16)] = rows_v[i, pl.ds(j * 16, 16)] * SCALE
            return c

        lax.fori_loop(0, CHUNK, scale_row, 0)

        pltpu.sync_copy(
            rows_v, out_hbm.at[pl.ds(wid * rows_per_w + g * CHUNK, CHUNK)]
        )
        return carry

    lax.fori_loop(0, n_chunks, chunk_body, 0)


def kernel(tokens, embedding_weight):
    bt, seq = tokens.shape
    b = bt * seq
    assert b % (NW * CHUNK) == 0
    n_chunks = b // (NW * CHUNK)
    tok2d = tokens.reshape(b // IDX_BLK, IDX_BLK).astype(jnp.int32)

    mesh = plsc.VectorSubcoreMesh(core_axis_name="c", subcore_axis_name="s")
    emb = functools.partial(
        pl.kernel,
        mesh=mesh,
        out_type=jax.ShapeDtypeStruct((b, EMB_D), jnp.float32),
        scratch_types=[
            pltpu.VMEM((NJ, IDX_BLK), jnp.int32),
            pltpu.VMEM((CHUNK, EMB_D), jnp.float32),
            pltpu.SemaphoreType.DMA,
        ],
        compiler_params=pltpu.CompilerParams(use_tc_tiling_on_sc=False),
    )(functools.partial(_emb_kernel_body, n_chunks))

    out = emb(tok2d, embedding_weight)
    return out.reshape(bt, seq, EMB_D)
